# initial kernel scaffold (unmeasured)
import jax
import jax.numpy as jnp
from jax import lax
from jax.experimental import pallas as pl
from jax.experimental.pallas import tpu as pltpu

N_DEV = 32
LOG2_N = 5


def _allreduce_sum(y):

    def body(y_ref, out_ref, recv_buf, send_sems, recv_sems):
        my_i = lax.axis_index("i")
        out_ref[...] = y_ref[...]
        for k in range(LOG2_N):
            partner = my_i ^ (1 << k)
            rdma = pltpu.make_async_remote_copy(
                src_ref=out_ref,
                dst_ref=recv_buf.at[k],
                send_sem=send_sems.at[k],
                recv_sem=recv_sems.at[k],
                device_id=(partner,),
                device_id_type=pl.DeviceIdType.MESH,
            )
            rdma.start()
            rdma.wait()
            out_ref[...] += recv_buf[k]

    return pl.pallas_call(
        body,
        out_shape=jax.ShapeDtypeStruct(y.shape, y.dtype),
        in_specs=[pl.BlockSpec(memory_space=pltpu.VMEM)],
        out_specs=pl.BlockSpec(memory_space=pltpu.VMEM),
        scratch_shapes=[
            pltpu.VMEM((LOG2_N,) + y.shape, y.dtype),
            pltpu.SemaphoreType.DMA((LOG2_N,)),
            pltpu.SemaphoreType.DMA((LOG2_N,)),
        ],
        compiler_params=pltpu.CompilerParams(collective_id=0),
    )(y)


def kernel(x, Wq, K_ext, V_ext, Wo):
    i = lax.axis_index("i")
    B, Sq, D = x.shape
    _, Skv, Hq_per, Dh = K_ext.shape
    dh = Hq_per * Dh

    bf16 = jnp.bfloat16
    Wq_my = lax.dynamic_slice(Wq, (0, i * dh), (D, dh))
    Q = (x.astype(bf16) @ Wq_my.astype(bf16)).reshape(B, Sq, Hq_per, Dh)

    scores = jnp.einsum(
        "bihd,bjhd->bhij", Q, K_ext.astype(bf16),
        preferred_element_type=jnp.float32,
    ) * 0.125
    w = jax.nn.softmax(scores, axis=-1)

    ctx = jnp.einsum(
        "bhij,bjhd->bihd", w.astype(bf16), V_ext.astype(bf16),
        preferred_element_type=jnp.float32,
    ).reshape(B, Sq, dh)

    Wo_my = lax.dynamic_slice(Wo, (i * dh, 0), (dh, Wo.shape[1]))
    y = jnp.matmul(
        ctx.astype(bf16), Wo_my.astype(bf16),
        preferred_element_type=jnp.float32,
    )

    return _allreduce_sum(y)


# baseline (device time: 70341 ns/iter reference)
import jax
import jax.numpy as jnp
from jax import lax
from jax.experimental import pallas as pl
from jax.experimental.pallas import tpu as pltpu

N_DEV = 32
LOG2_N = 5


def _allreduce_sum(y):

    def body(y_ref, out_ref, recv_buf, send_sems, recv_sems):
        my_i = lax.axis_index("i")
        out_ref[...] = y_ref[...]
        for k in range(LOG2_N):
            partner = my_i ^ (1 << k)
            rdma = pltpu.make_async_remote_copy(
                src_ref=out_ref,
                dst_ref=recv_buf.at[k],
                send_sem=send_sems.at[k],
                recv_sem=recv_sems.at[k],
                device_id=(partner,),
                device_id_type=pl.DeviceIdType.MESH,
            )
            rdma.start()
            rdma.wait()
            out_ref[...] += recv_buf[k]

    return pl.pallas_call(
        body,
        out_shape=jax.ShapeDtypeStruct(y.shape, y.dtype),
        in_specs=[pl.BlockSpec(memory_space=pltpu.VMEM)],
        out_specs=pl.BlockSpec(memory_space=pltpu.VMEM),
        scratch_shapes=[
            pltpu.VMEM((LOG2_N,) + y.shape, y.dtype),
            pltpu.SemaphoreType.DMA((LOG2_N,)),
            pltpu.SemaphoreType.DMA((LOG2_N,)),
        ],
    )(y)


def kernel(x, Wq, K_ext, V_ext, Wo):
    i = lax.axis_index("i")
    B, Sq, D = x.shape
    _, Skv, Hq_per, Dh = K_ext.shape
    dh = Hq_per * Dh

    bf16 = jnp.bfloat16
    Wq_my = lax.dynamic_slice(Wq, (0, i * dh), (D, dh))
    Q = (x.astype(bf16) @ Wq_my.astype(bf16)).reshape(B, Sq, Hq_per, Dh)

    scores = jnp.einsum(
        "bihd,bjhd->bhij", Q, K_ext.astype(bf16),
        preferred_element_type=jnp.float32,
    ) * 0.125
    w = jax.nn.softmax(scores, axis=-1)

    ctx = jnp.einsum(
        "bhij,bjhd->bihd", w.astype(bf16), V_ext.astype(bf16),
        preferred_element_type=jnp.float32,
    ).reshape(B, Sq, dh)

    Wo_my = lax.dynamic_slice(Wo, (i * dh, 0), (dh, Wo.shape[1]))
    y = jnp.matmul(
        ctx.astype(bf16), Wo_my.astype(bf16),
        preferred_element_type=jnp.float32,
    )

    return _allreduce_sum(y)


# device time: 39248 ns/iter; 1.7922x vs baseline; 1.7922x over previous
import jax
import jax.numpy as jnp
from jax import lax
from jax.experimental import pallas as pl
from jax.experimental.pallas import tpu as pltpu

N_DEV = 32


def _allreduce_sum(y):
    R, C = y.shape
    rows = R // N_DEV

    def body(y_ref, out_ref, acc_buf, p1_send, p1_recv, p2_send, p2_recv):
        my_i = lax.axis_index("i")

        p1 = []
        for d in range(1, N_DEV):
            partner = my_i ^ d
            rdma = pltpu.make_async_remote_copy(
                src_ref=y_ref.at[pl.ds(partner * rows, rows)],
                dst_ref=acc_buf.at[d],
                send_sem=p1_send.at[d],
                recv_sem=p1_recv.at[d],
                device_id=(partner,),
                device_id_type=pl.DeviceIdType.MESH,
            )
            rdma.start()
            p1.append(rdma)

        acc_buf[0, :, :] = y_ref[pl.ds(my_i * rows, rows), :]

        for rdma in p1:
            rdma.wait_recv()
        total = jnp.sum(acc_buf[...], axis=0)
        out_ref[pl.ds(my_i * rows, rows), :] = total

        p2 = []
        for d in range(1, N_DEV):
            partner = my_i ^ d
            rdma = pltpu.make_async_remote_copy(
                src_ref=out_ref.at[pl.ds(my_i * rows, rows)],
                dst_ref=out_ref.at[pl.ds(my_i * rows, rows)],
                send_sem=p2_send.at[d],
                recv_sem=p2_recv.at[d],
                device_id=(partner,),
                device_id_type=pl.DeviceIdType.MESH,
            )
            rdma.start()
            p2.append(rdma)

        for rdma in p2:
            rdma.wait_recv()
        for rdma in p1 + p2:
            rdma.wait_send()

    return pl.pallas_call(
        body,
        out_shape=jax.ShapeDtypeStruct((R, C), y.dtype),
        in_specs=[pl.BlockSpec(memory_space=pltpu.VMEM)],
        out_specs=pl.BlockSpec(memory_space=pltpu.VMEM),
        scratch_shapes=[
            pltpu.VMEM((N_DEV, rows, C), y.dtype),
            pltpu.SemaphoreType.DMA((N_DEV,)),
            pltpu.SemaphoreType.DMA((N_DEV,)),
            pltpu.SemaphoreType.DMA((N_DEV,)),
            pltpu.SemaphoreType.DMA((N_DEV,)),
        ],
    )(y)


def kernel(x, Wq, K_ext, V_ext, Wo):
    i = lax.axis_index("i")
    B, Sq, D = x.shape
    _, Skv, Hq_per, Dh = K_ext.shape
    dh = Hq_per * Dh

    bf16 = jnp.bfloat16
    Wq_my = lax.dynamic_slice(Wq, (0, i * dh), (D, dh))
    Q = (x.astype(bf16) @ Wq_my.astype(bf16)).reshape(B, Sq, Hq_per, Dh)

    scores = jnp.einsum(
        "bihd,bjhd->bhij", Q, K_ext.astype(bf16),
        preferred_element_type=jnp.float32,
    ) * 0.125
    w = jax.nn.softmax(scores, axis=-1)

    ctx = jnp.einsum(
        "bhij,bjhd->bihd", w.astype(bf16), V_ext.astype(bf16),
        preferred_element_type=jnp.float32,
    ).reshape(B, Sq, dh)

    Wo_my = lax.dynamic_slice(Wo, (i * dh, 0), (dh, Wo.shape[1]))
    y = jnp.matmul(
        ctx.astype(bf16), Wo_my.astype(bf16),
        preferred_element_type=jnp.float32,
    )

    out = _allreduce_sum(y.reshape(B * Sq, D))
    return out.reshape(B, Sq, D)
